# baseline (device time: 26340 ns/iter reference)
import jax
import jax.numpy as jnp
from jax import lax
from jax.experimental import pallas as pl
from jax.experimental.pallas import tpu as pltpu

T = 512
D = 1024
V_SHARD = 8192
V_TILE = 2048
N_TILES = V_SHARD // V_TILE


def kernel(x, W, labels):
    def body(x_ref, w_ref, lab_ref, out_ref, acc_ref, recv_ref, send_sem, recv_sem):
        step = pl.program_id(0)

        @pl.when(step == 0)
        def _():
            acc_ref[...] = jnp.zeros_like(acc_ref)

        my_y = lax.axis_index("y")

        xb = x_ref[...].astype(jnp.bfloat16)
        wb = w_ref[...].astype(jnp.bfloat16)
        logits = jnp.dot(xb, wb, preferred_element_type=jnp.float32)

        col = (
            lax.broadcasted_iota(jnp.int32, (T, V_TILE), 1)
            + step * V_TILE
            + my_y * V_SHARD
        )
        lab = lab_ref[...].reshape(T, 1)
        s_tile = jnp.sum(jnp.exp(logits), axis=1, keepdims=True)
        ll_tile = jnp.sum(
            jnp.where(col == lab, logits, 0.0), axis=1, keepdims=True
        )
        acc_ref[:, 0:1] = acc_ref[:, 0:1] + s_tile
        acc_ref[:, 1:2] = acc_ref[:, 1:2] + ll_tile

        @pl.when(step == N_TILES - 1)
        def _():
            my_x = lax.axis_index("x")
            my_z = lax.axis_index("z")
            nbr = (my_x, 1 - my_y, my_z)

            barrier = pltpu.get_barrier_semaphore()
            pl.semaphore_signal(
                barrier, inc=1, device_id=nbr,
                device_id_type=pl.DeviceIdType.MESH,
            )
            pl.semaphore_wait(barrier, 1)

            rdma = pltpu.make_async_remote_copy(
                src_ref=acc_ref,
                dst_ref=recv_ref,
                send_sem=send_sem,
                recv_sem=recv_sem,
                device_id=nbr,
                device_id_type=pl.DeviceIdType.MESH,
            )
            rdma.start()
            rdma.wait()

            s_tot = acc_ref[:, 0:1] + recv_ref[:, 0:1]
            ll_tot = acc_ref[:, 1:2] + recv_ref[:, 1:2]
            nll = jnp.log(s_tot) - ll_tot
            out_ref[...] = nll[:, 0]

    return pl.pallas_call(
        body,
        grid=(N_TILES,),
        out_shape=jax.ShapeDtypeStruct((T,), jnp.float32),
        in_specs=[
            pl.BlockSpec((T, D), lambda i: (0, 0)),
            pl.BlockSpec((D, V_TILE), lambda i: (0, i)),
            pl.BlockSpec((T,), lambda i: (0,)),
        ],
        out_specs=pl.BlockSpec((T,), lambda i: (0,)),
        scratch_shapes=[
            pltpu.VMEM((T, 2), jnp.float32),
            pltpu.VMEM((T, 2), jnp.float32),
            pltpu.SemaphoreType.DMA,
            pltpu.SemaphoreType.DMA,
        ],
        compiler_params=pltpu.CompilerParams(collective_id=0),
    )(x, W, labels)
